# baseline (speedup 1.0000x reference)
import functools

import jax
import jax.numpy as jnp
from jax.experimental import pallas as pl
from jax.experimental.pallas import tpu as pltpu

HEADS = 2


def _matmul_body(x_ref, w_ref, b_ref, o_ref, *, act):
    acc = jnp.dot(x_ref[...], w_ref[...], preferred_element_type=jnp.float32)
    acc = acc + b_ref[...]
    if act == "relu":
        acc = jnp.maximum(acc, 0.0)
    elif act == "leaky":
        acc = jnp.where(acc >= 0.0, acc, 0.2 * acc)
    o_ref[...] = acc


def _pallas_matmul(x, w, b=None, act=None, m_blk=1024):
    m, k = x.shape
    k2, n = w.shape
    assert k == k2
    if b is None:
        b = jnp.zeros((n,), jnp.float32)
    m_pad = (m + m_blk - 1) // m_blk * m_blk
    if m_pad != m:
        x = jnp.pad(x, ((0, m_pad - m), (0, 0)))
    out = pl.pallas_call(
        functools.partial(_matmul_body, act=act),
        grid=(m_pad // m_blk,),
        in_specs=[
            pl.BlockSpec((m_blk, k), lambda i: (i, 0)),
            pl.BlockSpec((k, n), lambda i: (0, 0)),
            pl.BlockSpec((n,), lambda i: (0,)),
        ],
        out_specs=pl.BlockSpec((m_blk, n), lambda i: (i, 0)),
        out_shape=jax.ShapeDtypeStruct((m_pad, n), jnp.float32),
    )(x, w, b)
    return out[:m]


def _gat_conv(x, src, dst, n, p):
    h = _pallas_matmul(x, p["W"])
    heads, c = p["att_src"].shape
    hr = h.reshape(n, heads, c)
    a_src = (hr * p["att_src"][None]).sum(-1)
    a_dst = (hr * p["att_dst"][None]).sum(-1)
    e = a_src[src] + a_dst[dst]
    e = jax.nn.leaky_relu(e, 0.2)
    e_max = jax.ops.segment_max(e, dst, num_segments=n)
    e_max = jnp.where(jnp.isfinite(e_max), e_max, 0.0)
    ex = jnp.exp(e - e_max[dst])
    denom = jax.ops.segment_sum(ex, dst, num_segments=n)
    alpha = ex / (denom[dst] + 1e-16)
    msg = hr[src] * alpha[:, :, None]
    out = jax.ops.segment_sum(msg, dst, num_segments=n)
    return out.reshape(n, heads * c) + p["bias"]


def kernel(x, edge_index, edge_attr, params):
    n = x.shape[0]
    src, dst = edge_index[0], edge_index[1]
    sl = jnp.arange(n, dtype=src.dtype)
    src_sl = jnp.concatenate([src, sl])
    dst_sl = jnp.concatenate([dst, sl])
    feat = x[:, 2:]
    h = _gat_conv(feat, src_sl, dst_sl, n, params["e_conv1"])
    h = jax.nn.relu(h)
    h = _pallas_matmul(h, params["lin1_W"], params["lin1_b"])
    h = _gat_conv(h, src_sl, dst_sl, n, params["e_conv2"])
    h = jax.nn.relu(h)
    embed = _pallas_matmul(h, params["lin2_W"], params["lin2_b"])
    xi = x[dst]
    xj = x[src]
    m = jnp.concatenate([xi, xj - xi], axis=-1)
    m = _pallas_matmul(m, params["ec_W1"], params["ec_b1"], act="relu")
    m = _pallas_matmul(m, params["ec_W2"], params["ec_b2"])
    agg = jax.ops.segment_max(m, dst, num_segments=n)
    agg = jnp.where(jnp.isfinite(agg), agg, 0.0)
    z = _gat_conv(agg + embed, src_sl, dst_sl, n, params["conv1"])
    z = jax.nn.relu(z)
    x_list = []
    for pc in params["convs"]:
        residual = z
        z = _gat_conv(z + embed, src_sl, dst_sl, n, pc)
        z = jax.nn.relu(z) + residual
        x_list.append(z)
    z = _pallas_matmul(z, params["reg_W"], params["reg_b"])
    for i, pc in enumerate(reversed(params["convs_rev"])):
        residual = z
        z = _gat_conv(z + embed + x_list[-i - 1], src_sl, dst_sl, n, pc)
        z = jax.nn.relu(z) + residual
    z = _gat_conv(z + embed, src_sl, dst_sl, n, params["conv1_rev"])
    z = jax.nn.relu(z)
    return _pallas_matmul(z, params["final_W"], params["final_b"])


# speedup vs baseline: 3.1348x; 3.1348x over previous
import functools

import jax
import jax.numpy as jnp
from jax import lax
from jax.experimental import pallas as pl
from jax.experimental.pallas import tpu as pltpu
from jax.experimental.pallas import tpu_sc as plsc

HEADS = 2
_NC, _NS = 2, 16
_NW = _NC * _NS


def _sc_gather_body(table_hbm, idx_hbm, out_hbm, idx_v, rows_v, sem,
                    *, b_per_w, chunk, nch):
    wid = lax.axis_index("s") * _NC + lax.axis_index("c")
    base = wid * b_per_w

    def step(i, carry):
        off = base + i * chunk
        pltpu.sync_copy(idx_hbm.at[pl.ds(off, chunk)], idx_v)
        pltpu.async_copy(table_hbm.at[idx_v], rows_v, sem).wait()
        pltpu.sync_copy(rows_v, out_hbm.at[pl.ds(off, chunk)])
        return carry

    lax.fori_loop(0, nch, step, 0)


def _sc_gather(table, idx, chunk=128):
    (m,) = idx.shape
    _, d = table.shape
    per_w = chunk * _NW
    m_pad = (m + per_w - 1) // per_w * per_w
    if m_pad != m:
        idx = jnp.concatenate([idx, jnp.zeros((m_pad - m,), idx.dtype)])
    b_per_w = m_pad // _NW
    nch = b_per_w // chunk
    mesh = plsc.VectorSubcoreMesh(core_axis_name="c", subcore_axis_name="s")
    fn = pl.kernel(
        functools.partial(_sc_gather_body, b_per_w=b_per_w, chunk=chunk,
                          nch=nch),
        out_type=jax.ShapeDtypeStruct((m_pad, d), jnp.float32),
        mesh=mesh,
        scratch_types=[
            pltpu.VMEM((chunk,), jnp.int32),
            pltpu.VMEM((chunk, d), jnp.float32),
            pltpu.SemaphoreType.DMA,
        ],
    )
    return fn(table, idx)[:m]


def _matmul_body(x_ref, w_ref, b_ref, o_ref, *, act):
    acc = jnp.dot(x_ref[...], w_ref[...], preferred_element_type=jnp.float32)
    acc = acc + b_ref[...]
    if act == "relu":
        acc = jnp.maximum(acc, 0.0)
    elif act == "leaky":
        acc = jnp.where(acc >= 0.0, acc, 0.2 * acc)
    o_ref[...] = acc


def _pallas_matmul(x, w, b=None, act=None, m_blk=1024):
    m, k = x.shape
    k2, n = w.shape
    assert k == k2
    if b is None:
        b = jnp.zeros((n,), jnp.float32)
    m_pad = (m + m_blk - 1) // m_blk * m_blk
    if m_pad != m:
        x = jnp.pad(x, ((0, m_pad - m), (0, 0)))
    out = pl.pallas_call(
        functools.partial(_matmul_body, act=act),
        grid=(m_pad // m_blk,),
        in_specs=[
            pl.BlockSpec((m_blk, k), lambda i: (i, 0)),
            pl.BlockSpec((k, n), lambda i: (0, 0)),
            pl.BlockSpec((n,), lambda i: (0,)),
        ],
        out_specs=pl.BlockSpec((m_blk, n), lambda i: (i, 0)),
        out_shape=jax.ShapeDtypeStruct((m_pad, n), jnp.float32),
    )(x, w, b)
    return out[:m]


def _gat_conv(x, src, dst, n, p):
    h = _pallas_matmul(x, p["W"])
    heads, c = p["att_src"].shape
    hr = h.reshape(n, heads, c)
    a_src = (hr * p["att_src"][None]).sum(-1)
    a_dst = (hr * p["att_dst"][None]).sum(-1)
    e = a_src[src] + a_dst[dst]
    e = jax.nn.leaky_relu(e, 0.2)
    e_max = jax.ops.segment_max(e, dst, num_segments=n)
    e_max = jnp.where(jnp.isfinite(e_max), e_max, 0.0)
    ex = jnp.exp(e - e_max[dst])
    denom = jax.ops.segment_sum(ex, dst, num_segments=n)
    alpha = ex / (denom[dst] + 1e-16)
    hs = _sc_gather(h, src)
    msg = (hs.reshape(-1, heads, c) * alpha[:, :, None]).reshape(-1, heads * c)
    out = jax.ops.segment_sum(msg, dst, num_segments=n)
    return out + p["bias"]


def kernel(x, edge_index, edge_attr, params):
    n = x.shape[0]
    src, dst = edge_index[0], edge_index[1]
    sl = jnp.arange(n, dtype=src.dtype)
    src_sl = jnp.concatenate([src, sl])
    dst_sl = jnp.concatenate([dst, sl])
    feat = x[:, 2:]
    h = _gat_conv(feat, src_sl, dst_sl, n, params["e_conv1"])
    h = jax.nn.relu(h)
    h = _pallas_matmul(h, params["lin1_W"], params["lin1_b"])
    h = _gat_conv(h, src_sl, dst_sl, n, params["e_conv2"])
    h = jax.nn.relu(h)
    embed = _pallas_matmul(h, params["lin2_W"], params["lin2_b"])
    xi = _sc_gather(x, dst)
    xj = _sc_gather(x, src)
    m = jnp.concatenate([xi, xj - xi], axis=-1)
    m = _pallas_matmul(m, params["ec_W1"], params["ec_b1"], act="relu")
    m = _pallas_matmul(m, params["ec_W2"], params["ec_b2"])
    agg = jax.ops.segment_max(m, dst, num_segments=n)
    agg = jnp.where(jnp.isfinite(agg), agg, 0.0)
    z = _gat_conv(agg + embed, src_sl, dst_sl, n, params["conv1"])
    z = jax.nn.relu(z)
    x_list = []
    for pc in params["convs"]:
        residual = z
        z = _gat_conv(z + embed, src_sl, dst_sl, n, pc)
        z = jax.nn.relu(z) + residual
        x_list.append(z)
    z = _pallas_matmul(z, params["reg_W"], params["reg_b"])
    for i, pc in enumerate(reversed(params["convs_rev"])):
        residual = z
        z = _gat_conv(z + embed + x_list[-i - 1], src_sl, dst_sl, n, pc)
        z = jax.nn.relu(z) + residual
    z = _gat_conv(z + embed, src_sl, dst_sl, n, params["conv1_rev"])
    z = jax.nn.relu(z)
    return _pallas_matmul(z, params["final_W"], params["final_b"])


# speedup vs baseline: 4.2528x; 1.3567x over previous
import functools

import jax
import jax.numpy as jnp
from jax import lax
from jax.experimental import pallas as pl
from jax.experimental.pallas import tpu as pltpu
from jax.experimental.pallas import tpu_sc as plsc

HEADS = 2
_NC, _NS = 2, 16
_NW = _NC * _NS


def _sc_gather_body(table_hbm, idx_hbm, out_hbm, idx_v, rows_v, sem,
                    *, b_per_w, chunk, nch):
    wid = lax.axis_index("s") * _NC + lax.axis_index("c")
    base = wid * b_per_w

    def step(i, carry):
        off = base + i * chunk
        pltpu.sync_copy(idx_hbm.at[pl.ds(off, chunk)], idx_v)
        pltpu.async_copy(table_hbm.at[idx_v], rows_v, sem).wait()
        pltpu.sync_copy(rows_v, out_hbm.at[pl.ds(off, chunk)])
        return carry

    lax.fori_loop(0, nch, step, 0)


def _sc_gather(table, idx, chunk=128):
    (m,) = idx.shape
    _, d = table.shape
    per_w = chunk * _NW
    m_pad = (m + per_w - 1) // per_w * per_w
    if m_pad != m:
        idx = jnp.concatenate([idx, jnp.zeros((m_pad - m,), idx.dtype)])
    b_per_w = m_pad // _NW
    nch = b_per_w // chunk
    mesh = plsc.VectorSubcoreMesh(core_axis_name="c", subcore_axis_name="s")
    fn = pl.kernel(
        functools.partial(_sc_gather_body, b_per_w=b_per_w, chunk=chunk,
                          nch=nch),
        out_type=jax.ShapeDtypeStruct((m_pad, d), jnp.float32),
        mesh=mesh,
        scratch_types=[
            pltpu.VMEM((chunk,), jnp.int32),
            pltpu.VMEM((chunk, d), jnp.float32),
            pltpu.SemaphoreType.DMA,
        ],
    )
    return fn(table, idx)[:m]


def _matmul_body(x_ref, w_ref, b_ref, o_ref, *, act):
    acc = jnp.dot(x_ref[...], w_ref[...], preferred_element_type=jnp.float32)
    acc = acc + b_ref[...]
    if act == "relu":
        acc = jnp.maximum(acc, 0.0)
    elif act == "leaky":
        acc = jnp.where(acc >= 0.0, acc, 0.2 * acc)
    o_ref[...] = acc


def _pallas_matmul(x, w, b=None, act=None, m_blk=1024):
    m, k = x.shape
    k2, n = w.shape
    assert k == k2
    if b is None:
        b = jnp.zeros((n,), jnp.float32)
    m_pad = (m + m_blk - 1) // m_blk * m_blk
    if m_pad != m:
        x = jnp.pad(x, ((0, m_pad - m), (0, 0)))
    out = pl.pallas_call(
        functools.partial(_matmul_body, act=act),
        grid=(m_pad // m_blk,),
        in_specs=[
            pl.BlockSpec((m_blk, k), lambda i: (i, 0)),
            pl.BlockSpec((k, n), lambda i: (0, 0)),
            pl.BlockSpec((n,), lambda i: (0,)),
        ],
        out_specs=pl.BlockSpec((m_blk, n), lambda i: (i, 0)),
        out_shape=jax.ShapeDtypeStruct((m_pad, n), jnp.float32),
    )(x, w, b)
    return out[:m]


def _attn_mm_body(p0_ref, p1_ref, h_ref, s_ref, b_ref, o_ref, acc_ref,
                  *, nk, c):
    k = pl.program_id(1)

    @pl.when(k == 0)
    def _():
        acc_ref[...] = jnp.zeros_like(acc_ref)

    a0 = jnp.dot(p0_ref[...], h_ref[:, :c], preferred_element_type=jnp.float32)
    a1 = jnp.dot(p1_ref[...], h_ref[:, c:], preferred_element_type=jnp.float32)
    acc_ref[...] += jnp.concatenate([a0, a1], axis=1)

    @pl.when(k == nk - 1)
    def _():
        o_ref[...] = acc_ref[...] * s_ref[...] + b_ref[...]


def _attn_matmul(p0, p1, h, scale, bias, m_blk=1000, k_blk=2048):
    n, kp = p0.shape
    d = h.shape[1]
    c = d // 2
    h_pad = jnp.pad(h, ((0, kp - h.shape[0]), (0, 0)))
    nm, nk = n // m_blk, kp // k_blk
    return pl.pallas_call(
        functools.partial(_attn_mm_body, nk=nk, c=c),
        grid=(nm, nk),
        in_specs=[
            pl.BlockSpec((m_blk, k_blk), lambda m, k: (m, k)),
            pl.BlockSpec((m_blk, k_blk), lambda m, k: (m, k)),
            pl.BlockSpec((k_blk, d), lambda m, k: (k, 0)),
            pl.BlockSpec((m_blk, d), lambda m, k: (m, 0)),
            pl.BlockSpec((d,), lambda m, k: (0,)),
        ],
        out_specs=pl.BlockSpec((m_blk, d), lambda m, k: (m, 0)),
        out_shape=jax.ShapeDtypeStruct((n, d), jnp.float32),
        scratch_shapes=[pltpu.VMEM((m_blk, d), jnp.float32)],
        compiler_params=pltpu.CompilerParams(
            dimension_semantics=("parallel", "arbitrary")),
    )(p0, p1, h_pad, scale, bias)


def _gat_conv(x, src, dst, n, p):
    h = _pallas_matmul(x, p["W"])
    heads, c = p["att_src"].shape
    hr = h.reshape(n, heads, c)
    a_src = (hr * p["att_src"][None]).sum(-1)
    a_dst = (hr * p["att_dst"][None]).sum(-1)
    e = a_src[src] + a_dst[dst]
    e = jax.nn.leaky_relu(e, 0.2)
    e_max = jax.ops.segment_max(e, dst, num_segments=n)
    e_max = jnp.where(jnp.isfinite(e_max), e_max, 0.0)
    ex = jnp.exp(e - e_max[dst])
    denom = jax.ops.segment_sum(ex, dst, num_segments=n)
    kp = (n + 2047) // 2048 * 2048
    p0 = jnp.zeros((n, kp), jnp.float32).at[dst, src].add(ex[:, 0])
    p1 = jnp.zeros((n, kp), jnp.float32).at[dst, src].add(ex[:, 1])
    scale = jnp.repeat(1.0 / (denom + 1e-16), c, axis=1)
    return _attn_matmul(p0, p1, h, scale, p["bias"])


def kernel(x, edge_index, edge_attr, params):
    n = x.shape[0]
    src, dst = edge_index[0], edge_index[1]
    sl = jnp.arange(n, dtype=src.dtype)
    src_sl = jnp.concatenate([src, sl])
    dst_sl = jnp.concatenate([dst, sl])
    feat = x[:, 2:]
    h = _gat_conv(feat, src_sl, dst_sl, n, params["e_conv1"])
    h = jax.nn.relu(h)
    h = _pallas_matmul(h, params["lin1_W"], params["lin1_b"])
    h = _gat_conv(h, src_sl, dst_sl, n, params["e_conv2"])
    h = jax.nn.relu(h)
    embed = _pallas_matmul(h, params["lin2_W"], params["lin2_b"])
    xi = _sc_gather(x, dst)
    xj = _sc_gather(x, src)
    m = jnp.concatenate([xi, xj - xi], axis=-1)
    m = _pallas_matmul(m, params["ec_W1"], params["ec_b1"], act="relu")
    m = _pallas_matmul(m, params["ec_W2"], params["ec_b2"])
    agg = jax.ops.segment_max(m, dst, num_segments=n)
    agg = jnp.where(jnp.isfinite(agg), agg, 0.0)
    z = _gat_conv(agg + embed, src_sl, dst_sl, n, params["conv1"])
    z = jax.nn.relu(z)
    x_list = []
    for pc in params["convs"]:
        residual = z
        z = _gat_conv(z + embed, src_sl, dst_sl, n, pc)
        z = jax.nn.relu(z) + residual
        x_list.append(z)
    z = _pallas_matmul(z, params["reg_W"], params["reg_b"])
    for i, pc in enumerate(reversed(params["convs_rev"])):
        residual = z
        z = _gat_conv(z + embed + x_list[-i - 1], src_sl, dst_sl, n, pc)
        z = jax.nn.relu(z) + residual
    z = _gat_conv(z + embed, src_sl, dst_sl, n, params["conv1_rev"])
    z = jax.nn.relu(z)
    return _pallas_matmul(z, params["final_W"], params["final_b"])


# speedup vs baseline: 24.3735x; 5.7311x over previous
import functools

import jax
import jax.numpy as jnp
from jax import lax
from jax.experimental import pallas as pl
from jax.experimental.pallas import tpu as pltpu
from jax.experimental.pallas import tpu_sc as plsc

HEADS = 2
_NC, _NS = 2, 16
_NW = _NC * _NS


def _sc_gather_body(table_hbm, idx_hbm, out_hbm, idx_v, rows_v, sem,
                    *, b_per_w, chunk, nch):
    wid = lax.axis_index("s") * _NC + lax.axis_index("c")
    base = wid * b_per_w

    def step(i, carry):
        off = base + i * chunk
        pltpu.sync_copy(idx_hbm.at[pl.ds(off, chunk)], idx_v)
        pltpu.async_copy(table_hbm.at[idx_v], rows_v, sem).wait()
        pltpu.sync_copy(rows_v, out_hbm.at[pl.ds(off, chunk)])
        return carry

    lax.fori_loop(0, nch, step, 0)


def _sc_gather(table, idx, chunk=128):
    (m,) = idx.shape
    _, d = table.shape
    per_w = chunk * _NW
    m_pad = (m + per_w - 1) // per_w * per_w
    if m_pad != m:
        idx = jnp.concatenate([idx, jnp.zeros((m_pad - m,), idx.dtype)])
    b_per_w = m_pad // _NW
    nch = b_per_w // chunk
    mesh = plsc.VectorSubcoreMesh(core_axis_name="c", subcore_axis_name="s")
    fn = pl.kernel(
        functools.partial(_sc_gather_body, b_per_w=b_per_w, chunk=chunk,
                          nch=nch),
        out_type=jax.ShapeDtypeStruct((m_pad, d), jnp.float32),
        mesh=mesh,
        scratch_types=[
            pltpu.VMEM((chunk,), jnp.int32),
            pltpu.VMEM((chunk, d), jnp.float32),
            pltpu.SemaphoreType.DMA,
        ],
    )
    return fn(table, idx)[:m]


def _matmul_body(x_ref, w_ref, b_ref, o_ref, *, act):
    acc = jnp.dot(x_ref[...], w_ref[...], preferred_element_type=jnp.float32)
    acc = acc + b_ref[...]
    if act == "relu":
        acc = jnp.maximum(acc, 0.0)
    elif act == "leaky":
        acc = jnp.where(acc >= 0.0, acc, 0.2 * acc)
    o_ref[...] = acc


def _pallas_matmul(x, w, b=None, act=None, m_blk=1024):
    m, k = x.shape
    k2, n = w.shape
    assert k == k2
    if b is None:
        b = jnp.zeros((n,), jnp.float32)
    m_pad = (m + m_blk - 1) // m_blk * m_blk
    if m_pad != m:
        x = jnp.pad(x, ((0, m_pad - m), (0, 0)))
    out = pl.pallas_call(
        functools.partial(_matmul_body, act=act),
        grid=(m_pad // m_blk,),
        in_specs=[
            pl.BlockSpec((m_blk, k), lambda i: (i, 0)),
            pl.BlockSpec((k, n), lambda i: (0, 0)),
            pl.BlockSpec((n,), lambda i: (0,)),
        ],
        out_specs=pl.BlockSpec((m_blk, n), lambda i: (i, 0)),
        out_shape=jax.ShapeDtypeStruct((m_pad, n), jnp.float32),
    )(x, w, b)
    return out[:m]


def _attn_mm_body(c_ref, st_ref, dt_ref, h_ref, b_ref, o_ref, acc_ref,
                  den_ref, *, nk, c):
    k = pl.program_id(1)

    @pl.when(k == 0)
    def _():
        acc_ref[...] = jnp.zeros_like(acc_ref)
        den_ref[...] = jnp.zeros_like(den_ref)

    cb = c_ref[...]
    p0 = cb * jnp.maximum(dt_ref[:, 0:1] * st_ref[0:1, :],
                          dt_ref[:, 2:3] * st_ref[2:3, :])
    p1 = cb * jnp.maximum(dt_ref[:, 1:2] * st_ref[1:2, :],
                          dt_ref[:, 3:4] * st_ref[3:4, :])
    a0 = jnp.dot(p0, h_ref[:, :c], preferred_element_type=jnp.float32)
    a1 = jnp.dot(p1, h_ref[:, c:], preferred_element_type=jnp.float32)
    acc_ref[...] += jnp.concatenate([a0, a1], axis=1)
    den_ref[:, 0:1] += jnp.sum(p0, axis=1, keepdims=True)
    den_ref[:, 1:2] += jnp.sum(p1, axis=1, keepdims=True)

    @pl.when(k == nk - 1)
    def _():
        inv0 = 1.0 / den_ref[:, 0:1]
        inv1 = 1.0 / den_ref[:, 1:2]
        o_ref[...] = jnp.concatenate(
            [acc_ref[:, :c] * inv0, acc_ref[:, c:] * inv1],
            axis=1) + b_ref[...]


def _attn_matmul(cm, st, dt, h, bias, m_blk=1000, k_blk=2048):
    n, kp = cm.shape
    d = h.shape[1]
    c = d // 2
    h_pad = jnp.pad(h, ((0, kp - h.shape[0]), (0, 0)))
    nm, nk = n // m_blk, kp // k_blk
    return pl.pallas_call(
        functools.partial(_attn_mm_body, nk=nk, c=c),
        grid=(nm, nk),
        in_specs=[
            pl.BlockSpec((m_blk, k_blk), lambda m, k: (m, k)),
            pl.BlockSpec((4, k_blk), lambda m, k: (0, k)),
            pl.BlockSpec((m_blk, 4), lambda m, k: (m, 0)),
            pl.BlockSpec((k_blk, d), lambda m, k: (k, 0)),
            pl.BlockSpec((d,), lambda m, k: (0,)),
        ],
        out_specs=pl.BlockSpec((m_blk, d), lambda m, k: (m, 0)),
        out_shape=jax.ShapeDtypeStruct((n, d), jnp.float32),
        scratch_shapes=[pltpu.VMEM((m_blk, d), jnp.float32),
                        pltpu.VMEM((m_blk, 4), jnp.float32)],
        compiler_params=pltpu.CompilerParams(
            dimension_semantics=("parallel", "arbitrary")),
    )(cm, st, dt, h_pad, bias)


def _gat_conv(x, cm, n, p):
    h = _pallas_matmul(x, p["W"])
    heads, c = p["att_src"].shape
    hr = h.reshape(n, heads, c)
    a_s = (hr * p["att_src"][None]).sum(-1)
    a_d = (hr * p["att_dst"][None]).sum(-1)
    ms = jnp.max(a_s, axis=0)
    kp = cm.shape[1]
    sp = jnp.pad(a_s - ms[None], ((0, kp - n), (0, 0)))
    es1 = jnp.exp(sp)
    es2 = jnp.exp(0.2 * sp)
    st = jnp.concatenate([es1, es2], axis=1).T
    msd = ms[None] + a_d
    ed1 = jnp.exp(jnp.minimum(0.8 * msd, 0.0))
    ed2 = jnp.exp(jnp.minimum(-0.8 * msd, 0.0))
    dt = jnp.concatenate([ed1, ed2], axis=1)
    return _attn_matmul(cm, st, dt, h, p["bias"])


def kernel(x, edge_index, edge_attr, params):
    n = x.shape[0]
    src, dst = edge_index[0], edge_index[1]
    sl = jnp.arange(n, dtype=src.dtype)
    src_sl = jnp.concatenate([src, sl])
    dst_sl = jnp.concatenate([dst, sl])
    kp = (n + 2047) // 2048 * 2048
    cm = jnp.zeros((n, kp), jnp.float32).at[dst_sl, src_sl].add(1.0)
    feat = x[:, 2:]
    h = _gat_conv(feat, cm, n, params["e_conv1"])
    h = jax.nn.relu(h)
    h = _pallas_matmul(h, params["lin1_W"], params["lin1_b"])
    h = _gat_conv(h, cm, n, params["e_conv2"])
    h = jax.nn.relu(h)
    embed = _pallas_matmul(h, params["lin2_W"], params["lin2_b"])
    xi = _sc_gather(x, dst)
    xj = _sc_gather(x, src)
    m = jnp.concatenate([xi, xj - xi], axis=-1)
    m = _pallas_matmul(m, params["ec_W1"], params["ec_b1"], act="relu")
    m = _pallas_matmul(m, params["ec_W2"], params["ec_b2"])
    agg = jax.ops.segment_max(m, dst, num_segments=n)
    agg = jnp.where(jnp.isfinite(agg), agg, 0.0)
    z = _gat_conv(agg + embed, cm, n, params["conv1"])
    z = jax.nn.relu(z)
    x_list = []
    for pc in params["convs"]:
        residual = z
        z = _gat_conv(z + embed, cm, n, pc)
        z = jax.nn.relu(z) + residual
        x_list.append(z)
    z = _pallas_matmul(z, params["reg_W"], params["reg_b"])
    for i, pc in enumerate(reversed(params["convs_rev"])):
        residual = z
        z = _gat_conv(z + embed + x_list[-i - 1], cm, n, pc)
        z = jax.nn.relu(z) + residual
    z = _gat_conv(z + embed, cm, n, params["conv1_rev"])
    z = jax.nn.relu(z)
    return _pallas_matmul(z, params["final_W"], params["final_b"])


# speedup vs baseline: 28.9444x; 1.1875x over previous
import functools

import jax
import jax.numpy as jnp
from jax import lax
from jax.experimental import pallas as pl
from jax.experimental.pallas import tpu as pltpu
from jax.experimental.pallas import tpu_sc as plsc

HEADS = 2
_NC, _NS = 2, 16
_NW = _NC * _NS


def _sc_gather_body(table_hbm, idx_hbm, out_hbm, idx_v, rows_v, sem,
                    *, b_per_w, chunk, nch):
    wid = lax.axis_index("s") * _NC + lax.axis_index("c")
    base = wid * b_per_w

    def step(i, carry):
        off = base + i * chunk
        pltpu.sync_copy(idx_hbm.at[pl.ds(off, chunk)], idx_v)
        pltpu.async_copy(table_hbm.at[idx_v], rows_v, sem).wait()
        pltpu.sync_copy(rows_v, out_hbm.at[pl.ds(off, chunk)])
        return carry

    lax.fori_loop(0, nch, step, 0)


def _sc_gather(table, idx, chunk=128):
    (m,) = idx.shape
    _, d = table.shape
    per_w = chunk * _NW
    m_pad = (m + per_w - 1) // per_w * per_w
    if m_pad != m:
        idx = jnp.concatenate([idx, jnp.zeros((m_pad - m,), idx.dtype)])
    b_per_w = m_pad // _NW
    nch = b_per_w // chunk
    mesh = plsc.VectorSubcoreMesh(core_axis_name="c", subcore_axis_name="s")
    fn = pl.kernel(
        functools.partial(_sc_gather_body, b_per_w=b_per_w, chunk=chunk,
                          nch=nch),
        out_type=jax.ShapeDtypeStruct((m_pad, d), jnp.float32),
        mesh=mesh,
        scratch_types=[
            pltpu.VMEM((chunk,), jnp.int32),
            pltpu.VMEM((chunk, d), jnp.float32),
            pltpu.SemaphoreType.DMA,
        ],
    )
    return fn(table, idx)[:m]


def _matmul_body(x_ref, w_ref, b_ref, o_ref, *, act):
    acc = jnp.dot(x_ref[...], w_ref[...], preferred_element_type=jnp.float32)
    acc = acc + b_ref[...]
    if act == "relu":
        acc = jnp.maximum(acc, 0.0)
    elif act == "leaky":
        acc = jnp.where(acc >= 0.0, acc, 0.2 * acc)
    o_ref[...] = acc


def _pallas_matmul(x, w, b=None, act=None, m_blk=1024):
    m, k = x.shape
    k2, n = w.shape
    assert k == k2
    if b is None:
        b = jnp.zeros((n,), jnp.float32)
    m_pad = (m + m_blk - 1) // m_blk * m_blk
    if m_pad != m:
        x = jnp.pad(x, ((0, m_pad - m), (0, 0)))
    out = pl.pallas_call(
        functools.partial(_matmul_body, act=act),
        grid=(m_pad // m_blk,),
        in_specs=[
            pl.BlockSpec((m_blk, k), lambda i: (i, 0)),
            pl.BlockSpec((k, n), lambda i: (0, 0)),
            pl.BlockSpec((n,), lambda i: (0,)),
        ],
        out_specs=pl.BlockSpec((m_blk, n), lambda i: (i, 0)),
        out_shape=jax.ShapeDtypeStruct((m_pad, n), jnp.float32),
    )(x, w, b)
    return out[:m]


def _attn_mm_body(c_ref, st_ref, dt_ref, h_ref, b_ref, o_ref, acc_ref,
                  den_ref, *, nk, c):
    k = pl.program_id(1)

    @pl.when(k == 0)
    def _():
        acc_ref[...] = jnp.zeros_like(acc_ref)
        den_ref[...] = jnp.zeros_like(den_ref)

    cb = c_ref[...]
    p0 = cb * jnp.maximum(dt_ref[:, 0:1] * st_ref[0:1, :],
                          dt_ref[:, 2:3] * st_ref[2:3, :])
    p1 = cb * jnp.maximum(dt_ref[:, 1:2] * st_ref[1:2, :],
                          dt_ref[:, 3:4] * st_ref[3:4, :])
    a0 = jnp.dot(p0, h_ref[:, :c], preferred_element_type=jnp.float32)
    a1 = jnp.dot(p1, h_ref[:, c:], preferred_element_type=jnp.float32)
    acc_ref[...] += jnp.concatenate([a0, a1], axis=1)
    den_ref[:, 0:1] += jnp.sum(p0, axis=1, keepdims=True)
    den_ref[:, 1:2] += jnp.sum(p1, axis=1, keepdims=True)

    @pl.when(k == nk - 1)
    def _():
        inv0 = 1.0 / den_ref[:, 0:1]
        inv1 = 1.0 / den_ref[:, 1:2]
        o_ref[...] = jnp.concatenate(
            [acc_ref[:, :c] * inv0, acc_ref[:, c:] * inv1],
            axis=1) + b_ref[...]


def _attn_matmul(cm, st, dt, h, bias, m_blk=1000, k_blk=2048):
    n, kp = cm.shape
    d = h.shape[1]
    c = d // 2
    h_pad = jnp.pad(h, ((0, kp - h.shape[0]), (0, 0)))
    nm, nk = n // m_blk, kp // k_blk
    return pl.pallas_call(
        functools.partial(_attn_mm_body, nk=nk, c=c),
        grid=(nm, nk),
        in_specs=[
            pl.BlockSpec((m_blk, k_blk), lambda m, k: (m, k)),
            pl.BlockSpec((4, k_blk), lambda m, k: (0, k)),
            pl.BlockSpec((m_blk, 4), lambda m, k: (m, 0)),
            pl.BlockSpec((k_blk, d), lambda m, k: (k, 0)),
            pl.BlockSpec((d,), lambda m, k: (0,)),
        ],
        out_specs=pl.BlockSpec((m_blk, d), lambda m, k: (m, 0)),
        out_shape=jax.ShapeDtypeStruct((n, d), jnp.float32),
        scratch_shapes=[pltpu.VMEM((m_blk, d), jnp.float32),
                        pltpu.VMEM((m_blk, 4), jnp.float32)],
        compiler_params=pltpu.CompilerParams(
            dimension_semantics=("parallel", "arbitrary")),
    )(cm, st, dt, h_pad, bias)


def _ec_mm_body(u_ref, v_ref, b1_ref, w_ref, b2_ref, o_ref):
    m1 = jnp.maximum(u_ref[...] + v_ref[...] + b1_ref[...], 0.0)
    o_ref[...] = jnp.dot(m1, w_ref[...],
                         preferred_element_type=jnp.float32) + b2_ref[...]


def _ec_matmul(u, v, b1, w, b2, m_blk=1000):
    m, k = u.shape
    n_out = w.shape[1]
    m_pad = (m + m_blk - 1) // m_blk * m_blk
    if m_pad != m:
        u = jnp.pad(u, ((0, m_pad - m), (0, 0)))
        v = jnp.pad(v, ((0, m_pad - m), (0, 0)))
    out = pl.pallas_call(
        _ec_mm_body,
        grid=(m_pad // m_blk,),
        in_specs=[
            pl.BlockSpec((m_blk, k), lambda i: (i, 0)),
            pl.BlockSpec((m_blk, k), lambda i: (i, 0)),
            pl.BlockSpec((k,), lambda i: (0,)),
            pl.BlockSpec((k, n_out), lambda i: (0, 0)),
            pl.BlockSpec((n_out,), lambda i: (0,)),
        ],
        out_specs=pl.BlockSpec((m_blk, n_out), lambda i: (i, 0)),
        out_shape=jax.ShapeDtypeStruct((m_pad, n_out), jnp.float32),
    )(u, v, b1, w, b2)
    return out[:m]


def _gat_conv(x, cm, n, p):
    h = _pallas_matmul(x, p["W"])
    heads, c = p["att_src"].shape
    hr = h.reshape(n, heads, c)
    a_s = (hr * p["att_src"][None]).sum(-1)
    a_d = (hr * p["att_dst"][None]).sum(-1)
    ms = jnp.max(a_s, axis=0)
    kp = cm.shape[1]
    sp = jnp.pad(a_s - ms[None], ((0, kp - n), (0, 0)))
    es1 = jnp.exp(sp)
    es2 = jnp.exp(0.2 * sp)
    st = jnp.concatenate([es1, es2], axis=1).T
    msd = ms[None] + a_d
    ed1 = jnp.exp(jnp.minimum(0.8 * msd, 0.0))
    ed2 = jnp.exp(jnp.minimum(-0.8 * msd, 0.0))
    dt = jnp.concatenate([ed1, ed2], axis=1)
    return _attn_matmul(cm, st, dt, h, p["bias"])


def kernel(x, edge_index, edge_attr, params):
    n = x.shape[0]
    src, dst = edge_index[0], edge_index[1]
    sl = jnp.arange(n, dtype=src.dtype)
    src_sl = jnp.concatenate([src, sl])
    dst_sl = jnp.concatenate([dst, sl])
    kp = (n + 2047) // 2048 * 2048
    cm = jnp.zeros((n, kp), jnp.float32).at[dst_sl, src_sl].add(1.0)
    feat = x[:, 2:]
    h = _gat_conv(feat, cm, n, params["e_conv1"])
    h = jax.nn.relu(h)
    h = _pallas_matmul(h, params["lin1_W"], params["lin1_b"])
    h = _gat_conv(h, cm, n, params["e_conv2"])
    h = jax.nn.relu(h)
    embed = _pallas_matmul(h, params["lin2_W"], params["lin2_b"])
    f = x.shape[1]
    u = _pallas_matmul(x, params["ec_W1"][:f] - params["ec_W1"][f:])
    v = _pallas_matmul(x, params["ec_W1"][f:])
    ne = src.shape[0]
    uv = _sc_gather(jnp.concatenate([u, v], axis=0),
                    jnp.concatenate([dst, src + n]))
    m = _ec_matmul(uv[:ne], uv[ne:], params["ec_b1"],
                   params["ec_W2"], params["ec_b2"])
    agg = jax.ops.segment_max(m, dst, num_segments=n)
    agg = jnp.where(jnp.isfinite(agg), agg, 0.0)
    z = _gat_conv(agg + embed, cm, n, params["conv1"])
    z = jax.nn.relu(z)
    x_list = []
    for pc in params["convs"]:
        residual = z
        z = _gat_conv(z + embed, cm, n, pc)
        z = jax.nn.relu(z) + residual
        x_list.append(z)
    z = _pallas_matmul(z, params["reg_W"], params["reg_b"])
    for i, pc in enumerate(reversed(params["convs_rev"])):
        residual = z
        z = _gat_conv(z + embed + x_list[-i - 1], cm, n, pc)
        z = jax.nn.relu(z) + residual
    z = _gat_conv(z + embed, cm, n, params["conv1_rev"])
    z = jax.nn.relu(z)
    return _pallas_matmul(z, params["final_W"], params["final_b"])
